# Initial kernel scaffold; baseline (speedup 1.0000x reference)
import jax
import jax.numpy as jnp
from jax.experimental import pallas as pl


def kernel(x, edge_index, edge_weight, eps0, W1, b1, g1, be1, eps1, W2, b2, g2, be2, eps2, W3, b3):
    raise NotImplementedError("write your pallas kernel here")


# speedup vs baseline: 4.9210x; 4.9210x over previous
import math

import jax
import jax.numpy as jnp
from jax import lax
from jax.experimental import pallas as pl
from jax.experimental.pallas import tpu as pltpu
from jax.experimental.pallas import tpu_sc as plsc

N = 10000
E = 320000
D = 128

NC = 2
NS = 16
NW = NC * NS
L = 16

CH = 128
ET = E + N
EPW_CH = (ET + NW * CH - 1) // (NW * CH)
EPW = EPW_CH * CH
EP = EPW * NW
NP = 10112
RPT = NP // NS
ZR = 128


def _sc_agg_body(h_hbm, src_hbm, dst_hbm, w_hbm, out_hbm,
                 src_v, dst_v, w_v, rows_v, acc_sh, sem):
    c = lax.axis_index("c")
    s = lax.axis_index("s")
    wid = c * NS + s

    pltpu.sync_copy(src_hbm.at[wid], src_v)
    pltpu.sync_copy(dst_hbm.at[wid], dst_v)
    pltpu.sync_copy(w_hbm.at[wid], w_v)

    zvec = jnp.zeros((L,), jnp.float32)

    def zero_row(r, carry):
        for j in range(D // L):
            rows_v[r, pl.ds(j * L, L)] = zvec
        return carry

    lax.fori_loop(0, ZR, zero_row, 0)
    for k in range(RPT // ZR):
        pltpu.sync_copy(rows_v, acc_sh.at[pl.ds(s * RPT + k * ZR, ZR)])
    rem = RPT - (RPT // ZR) * ZR
    if rem:
        pltpu.sync_copy(rows_v.at[pl.ds(0, rem)],
                        acc_sh.at[pl.ds(s * RPT + (RPT // ZR) * ZR, rem)])
    plsc.subcore_barrier()

    def chunk(i, carry):
        pltpu.async_copy(h_hbm.at[src_v.at[i]], rows_v, sem).wait()

        def group(g, cc):
            w16 = w_v[i, pl.ds(g * L, L)]
            for el in range(L):
                w = w16[el]
                e = g * L + el
                for j in range(D // L):
                    rows_v[e, pl.ds(j * L, L)] = rows_v[e, pl.ds(j * L, L)] * w
            return cc

        lax.fori_loop(0, CH // L, group, 0)
        pltpu.sync_copy(rows_v, acc_sh.at[dst_v.at[i]], add=True)
        return carry

    lax.fori_loop(0, EPW_CH, chunk, 0)
    plsc.subcore_barrier()

    for k in range(RPT // ZR):
        pltpu.sync_copy(acc_sh.at[pl.ds(s * RPT + k * ZR, ZR)], rows_v)
        pltpu.sync_copy(rows_v, out_hbm.at[c, pl.ds(s * RPT + k * ZR, ZR)])
    if RPT % ZR:
        k = RPT // ZR
        rem = RPT - k * ZR
        pltpu.sync_copy(acc_sh.at[pl.ds(s * RPT + k * ZR, rem)],
                        rows_v.at[pl.ds(0, rem)])
        pltpu.sync_copy(rows_v.at[pl.ds(0, rem)],
                        out_hbm.at[c, pl.ds(s * RPT + k * ZR, rem)])


_sc_agg = pl.kernel(
    _sc_agg_body,
    out_type=jax.ShapeDtypeStruct((NC, NP, D), jnp.float32),
    mesh=plsc.VectorSubcoreMesh(core_axis_name="c", subcore_axis_name="s"),
    scratch_types=[
        pltpu.VMEM((EPW_CH, CH), jnp.int32),
        pltpu.VMEM((EPW_CH, CH), jnp.int32),
        pltpu.VMEM((EPW_CH, CH), jnp.float32),
        pltpu.VMEM((CH, D), jnp.float32),
        pltpu.VMEM_SHARED((NP, D), jnp.float32),
        pltpu.SemaphoreType.DMA,
    ],
)


def _mlp_body(p_ref, wt_ref, scale_ref, bias_ref, out_ref):
    comb = p_ref[0] + p_ref[1]
    y = jnp.dot(comb, wt_ref[...], preferred_element_type=jnp.float32)
    out_ref[...] = jnp.maximum(y * scale_ref[...] + bias_ref[...], 0.0)


def _mlp(p, wt, scale, bias):
    rb = 1000
    return pl.pallas_call(
        _mlp_body,
        grid=(N // rb,),
        in_specs=[
            pl.BlockSpec((NC, rb, D), lambda i: (0, i, 0)),
            pl.BlockSpec((D, D), lambda i: (0, 0)),
            pl.BlockSpec((1, D), lambda i: (0, 0)),
            pl.BlockSpec((1, D), lambda i: (0, 0)),
        ],
        out_specs=pl.BlockSpec((rb, D), lambda i: (i, 0)),
        out_shape=jax.ShapeDtypeStruct((N, D), jnp.float32),
    )(p, wt, scale, bias)


def kernel(x, edge_index, edge_weight, eps0, W1, b1, g1, be1,
           eps1, W2, b2, g2, be2, eps2, W3, b3):
    src = edge_index[0]
    dst = edge_index[1]
    node_ids = jnp.arange(N, dtype=jnp.int32)
    pad = EP - ET
    src_p = jnp.concatenate(
        [src, node_ids, jnp.zeros((pad,), jnp.int32)]).reshape(NW, EPW_CH, CH)
    dst_p = jnp.concatenate(
        [dst, node_ids, jnp.zeros((pad,), jnp.int32)]).reshape(NW, EPW_CH, CH)

    def wts(eps):
        return jnp.concatenate([
            edge_weight,
            jnp.broadcast_to(1.0 + eps[0], (N,)).astype(jnp.float32),
            jnp.zeros((pad,), jnp.float32),
        ]).reshape(NW, EPW_CH, CH)

    bn_s = jnp.float32(1.0 / math.sqrt(1.0 + 1e-5))
    scale1 = (g1 * bn_s).reshape(1, D)
    bias1 = (b1 * g1 * bn_s + be1).reshape(1, D)
    scale2 = (g2 * bn_s).reshape(1, D)
    bias2 = (b2 * g2 * bn_s + be2).reshape(1, D)
    scale3 = jnp.ones((1, D), jnp.float32)
    bias3 = b3.reshape(1, D)

    h = x
    for (wcat, W, sc, bi) in ((wts(eps0), W1, scale1, bias1),
                              (wts(eps1), W2, scale2, bias2),
                              (wts(eps2), W3, scale3, bias3)):
        p = _sc_agg(h, src_p, dst_p, wcat)
        h = _mlp(p, W.T, sc, bi)
    return h
